# Initial kernel scaffold; baseline (speedup 1.0000x reference)
import jax
import jax.numpy as jnp
from jax.experimental import pallas as pl


def kernel(x, z, atomref_weight):
    raise NotImplementedError("write your pallas kernel here")


# speedup vs baseline: 79.8273x; 79.8273x over previous
import jax
import jax.numpy as jnp
from jax import lax
from jax.experimental import pallas as pl
from jax.experimental.pallas import tpu as pltpu
from jax.experimental.pallas import tpu_sc as plsc

N = 1_000_000
NW = 32
C = 31_264
STEPS = C // 16
TAB_PAD = 128


def _body(x_hbm, z_hbm, tab_hbm, out_hbm, tab_v, x_v, z_v):
    c = lax.axis_index("c")
    s = lax.axis_index("s")
    wid = s * 2 + c
    base = lax.min(wid * C, N - C)
    base = pl.multiple_of(base, 8)
    pltpu.sync_copy(tab_hbm, tab_v)
    pltpu.sync_copy(x_hbm.at[pl.ds(base, C)], x_v)
    pltpu.sync_copy(z_hbm.at[pl.ds(base, C)], z_v)

    def step(i, carry):
        sl = pl.ds(i * 16, 16)
        idx = z_v[sl]
        x_v[sl] = x_v[sl] + plsc.load_gather(tab_v, [idx])
        return carry

    lax.fori_loop(0, STEPS, step, 0)
    pltpu.sync_copy(x_v, out_hbm.at[pl.ds(base, C)])


def kernel(x, z, atomref_weight):
    xf = x.reshape(N)
    zi = z.astype(jnp.int32)
    tab = jnp.pad(atomref_weight.reshape(-1).astype(jnp.float32),
                  (0, TAB_PAD - atomref_weight.shape[0]))
    mesh = plsc.VectorSubcoreMesh(core_axis_name="c", subcore_axis_name="s")
    out = pl.kernel(
        _body,
        out_type=jax.ShapeDtypeStruct((N,), jnp.float32),
        mesh=mesh,
        compiler_params=pltpu.CompilerParams(needs_layout_passes=False),
        scratch_types=[
            pltpu.VMEM((TAB_PAD,), jnp.float32),
            pltpu.VMEM((C,), jnp.float32),
            pltpu.VMEM((C,), jnp.int32),
        ],
    )(xf, zi, tab)
    return out.reshape(N, 1)


# speedup vs baseline: 86.5826x; 1.0846x over previous
import jax
import jax.numpy as jnp
from jax import lax
from jax.experimental import pallas as pl
from jax.experimental.pallas import tpu as pltpu
from jax.experimental.pallas import tpu_sc as plsc

N = 1_000_000
NW = 32
C = 31_264
STEPS = C // 16
TAB_PAD = 128


def _body(x_hbm, z_hbm, tab_hbm, out_hbm, tab_v, x_v, z_v):
    c = lax.axis_index("c")
    s = lax.axis_index("s")
    wid = s * 2 + c
    base = lax.min(wid * C, N - C)
    base = pl.multiple_of(base, 8)
    pltpu.sync_copy(tab_hbm, tab_v)
    pltpu.sync_copy(x_hbm.at[pl.ds(base, C)], x_v)
    pltpu.sync_copy(z_hbm.at[pl.ds(base, C)], z_v)

    @plsc.parallel_loop(0, C, step=16, unroll=8)
    def _step(i):
        sl = pl.ds(i, 16)
        idx = z_v[sl]
        x_v[sl] = x_v[sl] + plsc.load_gather(tab_v, [idx])
    pltpu.sync_copy(x_v, out_hbm.at[pl.ds(base, C)])


def kernel(x, z, atomref_weight):
    xf = x.reshape(N)
    zi = z.astype(jnp.int32)
    tab = jnp.pad(atomref_weight.reshape(-1).astype(jnp.float32),
                  (0, TAB_PAD - atomref_weight.shape[0]))
    mesh = plsc.VectorSubcoreMesh(core_axis_name="c", subcore_axis_name="s")
    out = pl.kernel(
        _body,
        out_type=jax.ShapeDtypeStruct((N,), jnp.float32),
        mesh=mesh,
        compiler_params=pltpu.CompilerParams(needs_layout_passes=False),
        scratch_types=[
            pltpu.VMEM((TAB_PAD,), jnp.float32),
            pltpu.VMEM((C,), jnp.float32),
            pltpu.VMEM((C,), jnp.int32),
        ],
    )(xf, zi, tab)
    return out.reshape(N, 1)


# speedup vs baseline: 176.9400x; 2.0436x over previous
import jax
import jax.numpy as jnp
from jax import lax
from jax.experimental import pallas as pl
from jax.experimental.pallas import tpu as pltpu
from jax.experimental.pallas import tpu_sc as plsc

N = 1_000_000
NW = 32
C = 31_264
TAB_PAD = 128


def _body(z_hbm, tab_hbm, out_hbm, tab_v, z_v, g_v):
    c = lax.axis_index("c")
    s = lax.axis_index("s")
    wid = s * 2 + c
    base = lax.min(wid * C, N - C)
    base = pl.multiple_of(base, 8)
    pltpu.sync_copy(tab_hbm, tab_v)
    pltpu.sync_copy(z_hbm.at[pl.ds(base, C)], z_v)

    @plsc.parallel_loop(0, C, step=16, unroll=8)
    def _step(i):
        sl = pl.ds(i, 16)
        g_v[sl] = plsc.load_gather(tab_v, [z_v[sl]])

    pltpu.sync_copy(g_v, out_hbm.at[pl.ds(base, C)])


def kernel(x, z, atomref_weight):
    zi = z.astype(jnp.int32)
    tab = jnp.pad(atomref_weight.reshape(-1).astype(jnp.float32),
                  (0, TAB_PAD - atomref_weight.shape[0]))
    mesh = plsc.VectorSubcoreMesh(core_axis_name="c", subcore_axis_name="s")
    g = pl.kernel(
        _body,
        out_type=jax.ShapeDtypeStruct((N,), jnp.float32),
        mesh=mesh,
        compiler_params=pltpu.CompilerParams(needs_layout_passes=False),
        scratch_types=[
            pltpu.VMEM((TAB_PAD,), jnp.float32),
            pltpu.VMEM((C,), jnp.int32),
            pltpu.VMEM((C,), jnp.float32),
        ],
    )(zi, tab)
    return x + g.reshape(N, 1)


# speedup vs baseline: 182.2071x; 1.0298x over previous
import jax
import jax.numpy as jnp
from jax import lax
from jax.experimental import pallas as pl
from jax.experimental.pallas import tpu as pltpu
from jax.experimental.pallas import tpu_sc as plsc

N = 1_000_000
NW = 32
C = 31_264
TAB_PAD = 128


H = C // 2


def _body(z_hbm, tab_hbm, out_hbm, tab_v, z_v, g_v, sem_t, sem0, sem1, sem_o):
    c = lax.axis_index("c")
    s = lax.axis_index("s")
    wid = s * 2 + c
    base = lax.min(wid * C, N - C)
    base = pl.multiple_of(base, 8)
    h_t = pltpu.async_copy(tab_hbm, tab_v, sem_t)
    h0 = pltpu.async_copy(z_hbm.at[pl.ds(base, H)], z_v.at[pl.ds(0, H)], sem0)
    h1 = pltpu.async_copy(z_hbm.at[pl.ds(base + H, H)], z_v.at[pl.ds(H, H)], sem1)
    h_t.wait()
    h0.wait()

    @plsc.parallel_loop(0, H, step=16, unroll=8)
    def _step0(i):
        sl = pl.ds(i, 16)
        g_v[sl] = plsc.load_gather(tab_v, [z_v[sl]])

    h_o = pltpu.async_copy(g_v.at[pl.ds(0, H)], out_hbm.at[pl.ds(base, H)], sem_o)
    h1.wait()

    @plsc.parallel_loop(H, C, step=16, unroll=8)
    def _step1(i):
        sl = pl.ds(i, 16)
        g_v[sl] = plsc.load_gather(tab_v, [z_v[sl]])

    pltpu.sync_copy(g_v.at[pl.ds(H, H)], out_hbm.at[pl.ds(base + H, H)])
    h_o.wait()


def kernel(x, z, atomref_weight):
    zi = z.astype(jnp.int32)
    tab = jnp.pad(atomref_weight.reshape(-1).astype(jnp.float32),
                  (0, TAB_PAD - atomref_weight.shape[0]))
    mesh = plsc.VectorSubcoreMesh(core_axis_name="c", subcore_axis_name="s")
    g = pl.kernel(
        _body,
        out_type=jax.ShapeDtypeStruct((N,), jnp.float32),
        mesh=mesh,
        compiler_params=pltpu.CompilerParams(needs_layout_passes=False),
        scratch_types=[
            pltpu.VMEM((TAB_PAD,), jnp.float32),
            pltpu.VMEM((C,), jnp.int32),
            pltpu.VMEM((C,), jnp.float32),
            pltpu.SemaphoreType.DMA,
            pltpu.SemaphoreType.DMA,
            pltpu.SemaphoreType.DMA,
            pltpu.SemaphoreType.DMA,
        ],
    )(zi, tab)
    return x + g.reshape(N, 1)


# speedup vs baseline: 182.2245x; 1.0001x over previous
import jax
import jax.numpy as jnp
from jax import lax
from jax.experimental import pallas as pl
from jax.experimental.pallas import tpu as pltpu
from jax.experimental.pallas import tpu_sc as plsc

N = 1_000_000
NW = 32
C = 31_264
TAB_PAD = 128


H = C // 2


def _body(z_hbm, tab_hbm, out_hbm, tab_v, z_v, g_v, sem_t, sem0, sem1, sem_o):
    c = lax.axis_index("c")
    s = lax.axis_index("s")
    wid = s * 2 + c
    base = lax.min(wid * C, N - C)
    base = pl.multiple_of(base, 8)
    h_t = pltpu.async_copy(tab_hbm, tab_v, sem_t)
    h0 = pltpu.async_copy(z_hbm.at[pl.ds(base, H)], z_v.at[pl.ds(0, H)], sem0)
    h1 = pltpu.async_copy(z_hbm.at[pl.ds(base + H, H)], z_v.at[pl.ds(H, H)], sem1)
    h_t.wait()
    h0.wait()

    @plsc.parallel_loop(0, H, step=16, unroll=8)
    def _step0(i):
        sl = pl.ds(i, 16)
        g_v[sl] = plsc.load_gather(tab_v, [z_v[sl]])

    h_o = pltpu.async_copy(g_v.at[pl.ds(0, H)], out_hbm.at[pl.ds(base, H)], sem_o)
    h1.wait()

    @plsc.parallel_loop(H, C, step=16, unroll=8)
    def _step1(i):
        sl = pl.ds(i, 16)
        g_v[sl] = plsc.load_gather(tab_v, [z_v[sl]])

    pltpu.sync_copy(g_v.at[pl.ds(H, H)], out_hbm.at[pl.ds(base + H, H)])
    h_o.wait()


def kernel(x, z, atomref_weight):
    zi = z.astype(jnp.int32)
    tab = jnp.pad(atomref_weight.reshape(-1).astype(jnp.float32),
                  (0, TAB_PAD - atomref_weight.shape[0]))
    mesh = plsc.VectorSubcoreMesh(core_axis_name="c", subcore_axis_name="s")
    g = pl.kernel(
        _body,
        out_type=jax.ShapeDtypeStruct((N,), jnp.float32),
        mesh=mesh,
        compiler_params=pltpu.CompilerParams(
            needs_layout_passes=False, skip_device_barrier=True),
        scratch_types=[
            pltpu.VMEM((TAB_PAD,), jnp.float32),
            pltpu.VMEM((C,), jnp.int32),
            pltpu.VMEM((C,), jnp.float32),
            pltpu.SemaphoreType.DMA,
            pltpu.SemaphoreType.DMA,
            pltpu.SemaphoreType.DMA,
            pltpu.SemaphoreType.DMA,
        ],
    )(zi, tab)
    return x + g.reshape(N, 1)
